# Initial kernel scaffold; baseline (speedup 1.0000x reference)
import jax
import jax.numpy as jnp
from jax.experimental import pallas as pl


def kernel(prompt):
    raise NotImplementedError("write your pallas kernel here")


# speedup vs baseline: 5.3414x; 5.3414x over previous
import jax
import jax.numpy as jnp
from jax import lax
from jax.experimental import pallas as pl
from jax.experimental.pallas import tpu as pltpu

_B, _K, _T = 16, 8, 4096
_S = _T + _K
_SPECIAL = 2048.0


def _body(p_ref, out_ref, valid_ref):
    for k in range(_K):
        pieces = [jnp.full((1, 1, 1 + k), _SPECIAL, jnp.float32),
                  p_ref[:, k:k + 1, :]]
        if _K - 1 - k > 0:
            pieces.append(jnp.full((1, 1, _K - 1 - k), _SPECIAL, jnp.float32))
        out_ref[:, k:k + 1, :] = jnp.concatenate(pieces, axis=2)
    s = lax.broadcasted_iota(jnp.int32, (_K, _S), 1)
    kk = lax.broadcasted_iota(jnp.int32, (_K, _S), 0)
    valid_ref[...] = (s >= 1 + kk) & (s < 1 + kk + _T)


def kernel(prompt):
    out, valid = pl.pallas_call(
        _body,
        grid=(_B,),
        in_specs=[pl.BlockSpec((1, _K, _T), lambda b: (b, 0, 0))],
        out_specs=[
            pl.BlockSpec((1, _K, _S), lambda b: (b, 0, 0)),
            pl.BlockSpec((_K, _S), lambda b: (0, 0)),
        ],
        out_shape=[
            jax.ShapeDtypeStruct((_B, _K, _S), jnp.float32),
            jax.ShapeDtypeStruct((_K, _S), jnp.bool_),
        ],
    )(prompt)
    return out, valid
